# baseline (speedup 1.0000x reference)
import functools

import jax
import jax.numpy as jnp
from jax import lax
from jax.experimental import pallas as pl
from jax.experimental.pallas import tpu as pltpu
from jax.experimental.pallas import tpu_sc as plsc

BATCH = 1024
SEQ = 50
DIM = 64
TOTAL = BATCH * SEQ
NUM_WORKERS = 32
B_PER_W = BATCH // NUM_WORKERS
IDS_PER_W = B_PER_W * SEQ
CHUNK = 80
NCHUNK = IDS_PER_W // CHUNK

_mesh = plsc.VectorSubcoreMesh(core_axis_name="c", subcore_axis_name="s")


@functools.partial(
    pl.kernel,
    mesh=_mesh,
    out_type=jax.ShapeDtypeStruct((BATCH, SEQ, DIM), jnp.float32),
    scratch_types=[
        pltpu.VMEM((IDS_PER_W,), jnp.int32),
        pltpu.VMEM((IDS_PER_W, DIM), jnp.float32),
        pltpu.SemaphoreType.DMA,
    ],
    compiler_params=pltpu.CompilerParams(use_tc_tiling_on_sc=False),
)
def _emb_lookup(idx_hbm, table_hbm, out_hbm, idx_v, rows_v, sem):
    wid = lax.axis_index("s") * 2 + lax.axis_index("c")
    pltpu.sync_copy(idx_hbm.at[pl.ds(wid * IDS_PER_W, IDS_PER_W)], idx_v)
    copies = []
    for i in range(NCHUNK):
        off = i * CHUNK
        copies.append(
            pltpu.async_copy(
                table_hbm.at[idx_v.at[pl.ds(off, CHUNK)]],
                rows_v.at[pl.ds(off, CHUNK)],
                sem,
            )
        )
    for c in copies:
        c.wait()
    for b in range(B_PER_W):
        pltpu.sync_copy(
            rows_v.at[pl.ds(b * SEQ, SEQ)],
            out_hbm.at[wid * B_PER_W + b],
        )


def kernel(token_ids, embedding_lookup):
    idx = token_ids.reshape(-1).astype(jnp.int32)
    return _emb_lookup(idx, embedding_lookup)


# speedup vs baseline: 1.0319x; 1.0319x over previous
import functools

import jax
import jax.numpy as jnp
from jax import lax
from jax.experimental import pallas as pl
from jax.experimental.pallas import tpu as pltpu
from jax.experimental.pallas import tpu_sc as plsc

BATCH = 1024
SEQ = 50
SEQ_PAD = 56
DIM = 64
VOCAB = 1000
VOCAB_PAD = 1024
NUM_WORKERS = 32
D_BLOCKS = DIM // 8
B_BLOCKS = BATCH // 128
UNITS = D_BLOCKS * B_BLOCKS

_mesh = plsc.VectorSubcoreMesh(core_axis_name="c", subcore_axis_name="s")


@functools.partial(
    pl.kernel,
    mesh=_mesh,
    out_type=jax.ShapeDtypeStruct((SEQ, DIM, BATCH), jnp.float32),
    scratch_types=[
        pltpu.VMEM((DIM, VOCAB_PAD), jnp.float32),
        pltpu.VMEM((SEQ_PAD, 256), jnp.int32),
        pltpu.VMEM((2, 8, 128), jnp.float32),
        pltpu.SemaphoreType.DMA,
        pltpu.SemaphoreType.DMA,
    ],
    compiler_params=pltpu.CompilerParams(
        use_tc_tiling_on_sc=True, needs_layout_passes=False
    ),
)
def _emb_lookup(ids_hbm, table_hbm, out_hbm, tab_v, ids_v, tile_v, sem, osem):
    wid = lax.axis_index("s") * 2 + lax.axis_index("c")
    unit0 = wid * 2
    dblk = unit0 // B_BLOCKS
    bblk0 = unit0 % B_BLOCKS
    tcopy = pltpu.async_copy(table_hbm, tab_v, sem)
    pltpu.sync_copy(ids_hbm.at[:, pl.ds(bblk0 * 128, 256)], ids_v)
    tcopy.wait()

    def make_tile(u, s, buf):
        for v in range(8):
            idx16 = ids_v[s, pl.ds(u * 128 + v * 16, 16)]
            for d8 in range(8):
                drow = jnp.full((16,), dblk * 8 + d8, jnp.int32)
                tile_v[buf, d8, pl.ds(v * 16, 16)] = plsc.load_gather(
                    tab_v, [drow, idx16]
                )

    for u in range(2):
        @pl.loop(0, SEQ)
        def seq_body(s):
            buf = s % 2
            make_tile(u, s, buf)
            pltpu.async_copy(
                tile_v.at[buf],
                out_hbm.at[s, pl.ds(dblk * 8, 8), pl.ds((bblk0 + u) * 128, 128)],
                osem,
            ).wait()


def kernel(token_ids, embedding_lookup):
    ids_t = jnp.pad(token_ids.astype(jnp.int32).T, ((0, SEQ_PAD - SEQ), (0, 0)))
    tab_t = jnp.pad(embedding_lookup.T, ((0, 0), (0, VOCAB_PAD - VOCAB)))
    out = _emb_lookup(ids_t, tab_t)
    return jnp.transpose(out, (2, 0, 1))


# speedup vs baseline: 1.1461x; 1.1107x over previous
import functools

import jax
import jax.numpy as jnp
from jax import lax
from jax.experimental import pallas as pl
from jax.experimental.pallas import tpu as pltpu
from jax.experimental.pallas import tpu_sc as plsc

BATCH = 1024
SEQ = 50
SEQ_PAD = 56
DIM = 64
VOCAB = 1000
VOCAB_PAD = 1024
NUM_WORKERS = 32
D_BLOCKS = DIM // 8
B_BLOCKS = BATCH // 128

_mesh = plsc.VectorSubcoreMesh(core_axis_name="c", subcore_axis_name="s")


@functools.partial(
    pl.kernel,
    mesh=_mesh,
    out_type=jax.ShapeDtypeStruct((SEQ, DIM, BATCH), jnp.float32),
    scratch_types=[
        pltpu.VMEM((DIM, VOCAB_PAD), jnp.float32),
        pltpu.VMEM((SEQ_PAD, 256), jnp.int32),
        pltpu.VMEM((2, 8, 128), jnp.float32),
        pltpu.SemaphoreType.DMA,
        pltpu.SemaphoreType.DMA,
        pltpu.SemaphoreType.DMA,
    ],
    compiler_params=pltpu.CompilerParams(
        use_tc_tiling_on_sc=True, needs_layout_passes=False
    ),
)
def _emb_lookup(ids_hbm, table_hbm, out_hbm, tab_v, ids_v, tile_v, sem, os0, os1):
    wid = lax.axis_index("s") * 2 + lax.axis_index("c")
    unit0 = wid * 2
    dblk = unit0 // B_BLOCKS
    bblk0 = unit0 % B_BLOCKS
    osem = (os0, os1)
    tcopy = pltpu.async_copy(table_hbm, tab_v, sem)
    pltpu.sync_copy(ids_hbm.at[:, pl.ds(bblk0 * 128, 256)], ids_v)
    tcopy.wait()

    def make_tile(u, s, buf):
        for v in range(8):
            idx16 = ids_v[s, pl.ds(u * 128 + v * 16, 16)]
            for d8 in range(8):
                drow = jnp.full((16,), dblk * 8 + d8, jnp.int32)
                tile_v[buf, d8, pl.ds(v * 16, 16)] = plsc.load_gather(
                    tab_v, [drow, idx16]
                )

    def dst(u, s):
        return out_hbm.at[s, pl.ds(dblk * 8, 8), pl.ds((bblk0 + u) * 128, 128)]

    for u in range(2):
        for b in range(2):
            make_tile(u, b, b)
            pltpu.async_copy(tile_v.at[b], dst(u, b), osem[b])

        @pl.loop(2, SEQ, step=2)
        def seq_body(s0):
            for b in range(2):
                s = s0 + b
                pltpu.make_async_copy(tile_v.at[b], dst(u, s - 2), osem[b]).wait()
                make_tile(u, s, b)
                pltpu.async_copy(tile_v.at[b], dst(u, s), osem[b])

        for b in range(2):
            pltpu.make_async_copy(tile_v.at[b], dst(u, SEQ - 2 + b), osem[b]).wait()


def kernel(token_ids, embedding_lookup):
    ids_t = jnp.pad(token_ids.astype(jnp.int32).T, ((0, SEQ_PAD - SEQ), (0, 0)))
    tab_t = jnp.pad(embedding_lookup.T, ((0, 0), (0, VOCAB_PAD - VOCAB)))
    out = _emb_lookup(ids_t, tab_t)
    return jnp.transpose(out, (2, 0, 1))


# speedup vs baseline: 1.5486x; 1.3512x over previous
import functools

import jax
import jax.numpy as jnp
from jax import lax
from jax.experimental import pallas as pl
from jax.experimental.pallas import tpu as pltpu
from jax.experimental.pallas import tpu_sc as plsc

BATCH = 1024
SEQ = 50
SEQ_PAD = 56
DIM = 64
VOCAB = 1000
VOCAB_PAD = 1024
NUM_WORKERS = 32
D_BLOCKS = DIM // 8
B_BLOCKS = BATCH // 128

_mesh = plsc.VectorSubcoreMesh(core_axis_name="c", subcore_axis_name="s")


@functools.partial(
    pl.kernel,
    mesh=_mesh,
    out_type=jax.ShapeDtypeStruct((SEQ, DIM, BATCH), jnp.float32),
    scratch_types=[
        pltpu.VMEM((DIM, VOCAB_PAD), jnp.float32),
        pltpu.VMEM((SEQ_PAD, 256), jnp.int32),
        pltpu.VMEM((2, 8, 128), jnp.float32),
        pltpu.SemaphoreType.DMA,
        pltpu.SemaphoreType.DMA,
        pltpu.SemaphoreType.DMA,
    ],
    compiler_params=pltpu.CompilerParams(
        use_tc_tiling_on_sc=True, needs_layout_passes=False
    ),
)
def _emb_lookup(ids_hbm, table_hbm, out_hbm, tab_v, ids_v, tile_v, sem, os0, os1):
    wid = lax.axis_index("s") * 2 + lax.axis_index("c")
    unit0 = wid * 2
    dblk = unit0 // B_BLOCKS
    bblk0 = unit0 % B_BLOCKS
    osem = (os0, os1)
    tcopy = pltpu.async_copy(table_hbm, tab_v, sem)
    pltpu.sync_copy(ids_hbm.at[:, pl.ds(bblk0 * 128, 256)], ids_v)
    tcopy.wait()

    def make_tile(u, s, buf):
        for v in range(8):
            idx16 = ids_v[s, pl.ds(u * 128 + v * 16, 16)]
            gathered = [
                plsc.load_gather(
                    tab_v,
                    [jnp.full((16,), dblk * 8 + d8, jnp.int32), idx16],
                )
                for d8 in range(8)
            ]
            for d8 in range(8):
                tile_v[buf, d8, pl.ds(v * 16, 16)] = gathered[d8]

    def dst(u, s):
        return out_hbm.at[s, pl.ds(dblk * 8, 8), pl.ds((bblk0 + u) * 128, 128)]

    for u in range(2):
        for b in range(2):
            make_tile(u, b, b)
            pltpu.async_copy(tile_v.at[b], dst(u, b), osem[b])

        @pl.loop(2, SEQ, step=2)
        def seq_body(s0):
            for b in range(2):
                s = s0 + b
                pltpu.make_async_copy(tile_v.at[b], dst(u, s - 2), osem[b]).wait()
                make_tile(u, s, b)
                pltpu.async_copy(tile_v.at[b], dst(u, s), osem[b])

        for b in range(2):
            pltpu.make_async_copy(tile_v.at[b], dst(u, SEQ - 2 + b), osem[b]).wait()


def kernel(token_ids, embedding_lookup):
    ids_t = jnp.pad(token_ids.astype(jnp.int32).T, ((0, SEQ_PAD - SEQ), (0, 0)))
    tab_t = jnp.pad(embedding_lookup.T, ((0, 0), (0, VOCAB_PAD - VOCAB)))
    out = _emb_lookup(ids_t, tab_t)
    return jnp.transpose(out, (2, 0, 1))
